# Initial kernel scaffold; baseline (speedup 1.0000x reference)
import jax
import jax.numpy as jnp
from jax.experimental import pallas as pl


def kernel(z, emb, W1, b1, g_ln, b_ln, W2, b2):
    raise NotImplementedError("write your pallas kernel here")


# speedup vs baseline: 1.1721x; 1.1721x over previous
import functools

import jax
import jax.numpy as jnp
from jax.experimental import pallas as pl

B, T, V, C = 32, 256, 32, 128
N_E = 1024
HID = 256
BETA = 0.25
R = B * T
RB = 512
NRB = R // RB

WAVES = (
    ((0, 0),),
    ((1, 0), (6, 0), (11, 0)),
    ((2, 1), (7, 6), (12, 11)),
    ((3, 2), (8, 7), (13, 12), (16, 12), (24, 12)),
    ((4, 3), (9, 8), (14, 13), (17, 16), (25, 24)),
    ((5, 4), (10, 9), (15, 14), (18, 17), (26, 25)),
    ((19, 18), (27, 26)),
    ((20, 19), (28, 27)),
    ((21, 20), (29, 28)),
    ((22, 21), (30, 29)),
    ((23, 22), (31, 30)),
)


def _vq_tail(h, E, q_ref, e_ref, idx_ref, loss_ref, r):
    hn = jnp.sum(h * h, axis=1, keepdims=True)
    en = jnp.sum(E * E, axis=1)[None, :]
    d2 = hn - 2.0 * jnp.dot(h, E.T, preferred_element_type=jnp.float32) + en
    idx = jnp.argmin(d2, axis=1).astype(jnp.int32)
    oh = (jax.lax.broadcasted_iota(jnp.int32, (RB, N_E), 1) == idx[:, None])
    e = jnp.dot(oh.astype(jnp.float32), E, preferred_element_type=jnp.float32)
    diff = e - h
    q_ref[0] = h + diff
    e_ref[0] = e
    idx_ref[0, 0] = idx
    part = jnp.sum(diff * diff, axis=0, keepdims=True)[None]

    @pl.when(r == 0)
    def _():
        loss_ref[...] = part

    @pl.when(r != 0)
    def _():
        loss_ref[...] += part


def _root_body(zj_ref, E_ref, q_ref, e_ref, idx_ref, loss_ref):
    r = pl.program_id(1)
    _vq_tail(zj_ref[0], E_ref[0], q_ref, e_ref, idx_ref, loss_ref, r)


def _wave_body(p_ref, zj_ref, E_ref, W1a_ref, W1b_ref, b1_ref, g_ref,
               bl_ref, W2_ref, b2_ref, q_ref, e_ref, idx_ref, loss_ref):
    r = pl.program_id(1)
    h1 = (jnp.dot(p_ref[0], W1a_ref[...], preferred_element_type=jnp.float32)
          + jnp.dot(zj_ref[0], W1b_ref[...], preferred_element_type=jnp.float32)
          + b1_ref[...])
    m = jnp.mean(h1, axis=-1, keepdims=True)
    v = jnp.mean((h1 - m) ** 2, axis=-1, keepdims=True)
    h1 = (h1 - m) / jnp.sqrt(v + 1e-5) * g_ref[...] + bl_ref[...]
    h1 = jnp.maximum(h1, 0.0)
    h = jnp.dot(h1, W2_ref[...], preferred_element_type=jnp.float32) + b2_ref[...]
    _vq_tail(h, E_ref[0], q_ref, e_ref, idx_ref, loss_ref, r)


def _out_specs(nj):
    out_shape = (
        jax.ShapeDtypeStruct((nj, R, C), jnp.float32),
        jax.ShapeDtypeStruct((nj, R, C), jnp.float32),
        jax.ShapeDtypeStruct((nj * NRB, 1, RB), jnp.int32),
        jax.ShapeDtypeStruct((nj, 1, C), jnp.float32),
    )
    out_specs = (
        pl.BlockSpec((1, RB, C), lambda j, r: (j, r, 0)),
        pl.BlockSpec((1, RB, C), lambda j, r: (j, r, 0)),
        pl.BlockSpec((1, 1, RB), lambda j, r: (j * NRB + r, 0, 0)),
        pl.BlockSpec((1, 1, C), lambda j, r: (j, 0, 0)),
    )
    return out_shape, out_specs


@functools.lru_cache(maxsize=None)
def _root_call(nj):
    out_shape, out_specs = _out_specs(nj)
    return pl.pallas_call(
        _root_body,
        grid=(nj, NRB),
        in_specs=[
            pl.BlockSpec((1, RB, C), lambda j, r: (j, r, 0)),
            pl.BlockSpec((1, N_E, C), lambda j, r: (j, 0, 0)),
        ],
        out_specs=out_specs,
        out_shape=out_shape,
    )


@functools.lru_cache(maxsize=None)
def _wave_call(nj):
    out_shape, out_specs = _out_specs(nj)
    full = lambda j, r: (0, 0)
    return pl.pallas_call(
        _wave_body,
        grid=(nj, NRB),
        in_specs=[
            pl.BlockSpec((1, RB, C), lambda j, r: (j, r, 0)),
            pl.BlockSpec((1, RB, C), lambda j, r: (j, r, 0)),
            pl.BlockSpec((1, N_E, C), lambda j, r: (j, 0, 0)),
            pl.BlockSpec((C, HID), full),
            pl.BlockSpec((C, HID), full),
            pl.BlockSpec((1, HID), full),
            pl.BlockSpec((1, HID), full),
            pl.BlockSpec((1, HID), full),
            pl.BlockSpec((HID, C), full),
            pl.BlockSpec((1, C), full),
        ],
        out_specs=out_specs,
        out_shape=out_shape,
    )


def kernel(z, emb, W1, b1, g_ln, b_ln, W2, b2):
    zt = jnp.transpose(z, (2, 0, 1, 3)).reshape(V, R, C)
    W1a, W1b = W1[:C], W1[C:]
    b1r = b1.reshape(1, HID)
    gr = g_ln.reshape(1, HID)
    blr = b_ln.reshape(1, HID)
    b2r = b2.reshape(1, C)

    q_all = [None] * V
    e_all = [None] * V
    i_all = [None] * V
    loss_sum = jnp.float32(0.0)
    for w, wave in enumerate(WAVES):
        joints = jnp.array([j for j, _ in wave])
        nj = len(wave)
        zw = zt[joints]
        Ew = emb[joints]
        if w == 0:
            q, e, idx, lp = _root_call(nj)(zw, Ew)
        else:
            pw = jnp.stack([e_all[p] for _, p in wave])
            q, e, idx, lp = _wave_call(nj)(pw, zw, Ew, W1a, W1b, b1r, gr,
                                           blr, W2, b2r)
        idx = idx.reshape(nj, R)
        for k, (j, _) in enumerate(wave):
            q_all[j] = q[k]
            e_all[j] = e[k]
            i_all[j] = idx[k]
        loss_sum = loss_sum + jnp.sum(lp)

    z_q = jnp.stack(q_all, axis=0).reshape(V, B, T, C).transpose(1, 2, 0, 3)
    indices = jnp.stack(i_all, axis=0).reshape(V, B, T).transpose(1, 2, 0)
    total = (1.0 + BETA) * loss_sum / (V * R * C)
    return z_q, total, indices


# speedup vs baseline: 1.1761x; 1.0034x over previous
import functools

import jax
import jax.numpy as jnp
from jax.experimental import pallas as pl

B, T, V, C = 32, 256, 32, 128
N_E = 1024
HID = 256
BETA = 0.25
R = B * T
RB = 512
NRB = R // RB

WAVES = (
    ((0, 0),),
    ((1, 0), (6, 0), (11, 0)),
    ((2, 1), (7, 6), (12, 11)),
    ((3, 2), (8, 7), (13, 12), (16, 12), (24, 12)),
    ((4, 3), (9, 8), (14, 13), (17, 16), (25, 24)),
    ((5, 4), (10, 9), (15, 14), (18, 17), (26, 25)),
    ((19, 18), (27, 26)),
    ((20, 19), (28, 27)),
    ((21, 20), (29, 28)),
    ((22, 21), (30, 29)),
    ((23, 22), (31, 30)),
)


def _vq_tail(h, E, e_ref, idx_ref, loss_ref, r):
    hn = jnp.sum(h * h, axis=1, keepdims=True)
    en = jnp.sum(E * E, axis=1)[None, :]
    d2 = hn - 2.0 * jnp.dot(h, E.T, preferred_element_type=jnp.float32) + en
    idx = jnp.argmin(d2, axis=1).astype(jnp.int32)
    oh = (jax.lax.broadcasted_iota(jnp.int32, (RB, N_E), 1) == idx[:, None])
    e = jnp.dot(oh.astype(jnp.float32), E, preferred_element_type=jnp.float32)
    diff = e - h
    e_ref[0] = e
    idx_ref[0, 0] = idx
    part = jnp.sum(diff * diff, axis=0, keepdims=True)[None]

    @pl.when(r == 0)
    def _():
        loss_ref[...] = part

    @pl.when(r != 0)
    def _():
        loss_ref[...] += part


def _root_body(zj_ref, E_ref, e_ref, idx_ref, loss_ref):
    r = pl.program_id(1)
    _vq_tail(zj_ref[0], E_ref[0], e_ref, idx_ref, loss_ref, r)


def _wave_body(p_ref, zj_ref, E_ref, W1a_ref, W1b_ref, b1_ref, g_ref,
               bl_ref, W2_ref, b2_ref, e_ref, idx_ref, loss_ref):
    r = pl.program_id(1)
    h1 = (jnp.dot(p_ref[0], W1a_ref[...], preferred_element_type=jnp.float32)
          + jnp.dot(zj_ref[0], W1b_ref[...], preferred_element_type=jnp.float32)
          + b1_ref[...])
    m = jnp.mean(h1, axis=-1, keepdims=True)
    v = jnp.mean((h1 - m) ** 2, axis=-1, keepdims=True)
    h1 = (h1 - m) / jnp.sqrt(v + 1e-5) * g_ref[...] + bl_ref[...]
    h1 = jnp.maximum(h1, 0.0)
    h = jnp.dot(h1, W2_ref[...], preferred_element_type=jnp.float32) + b2_ref[...]
    _vq_tail(h, E_ref[0], e_ref, idx_ref, loss_ref, r)


def _out_specs(nj):
    out_shape = (
        jax.ShapeDtypeStruct((nj, R, C), jnp.float32),
        jax.ShapeDtypeStruct((nj * NRB, 1, RB), jnp.int32),
        jax.ShapeDtypeStruct((nj, 1, C), jnp.float32),
    )
    out_specs = (
        pl.BlockSpec((1, RB, C), lambda j, r: (j, r, 0)),
        pl.BlockSpec((1, 1, RB), lambda j, r: (j * NRB + r, 0, 0)),
        pl.BlockSpec((1, 1, C), lambda j, r: (j, 0, 0)),
    )
    return out_shape, out_specs


@functools.lru_cache(maxsize=None)
def _root_call(nj):
    out_shape, out_specs = _out_specs(nj)
    return pl.pallas_call(
        _root_body,
        grid=(nj, NRB),
        in_specs=[
            pl.BlockSpec((1, RB, C), lambda j, r: (j, r, 0)),
            pl.BlockSpec((1, N_E, C), lambda j, r: (j, 0, 0)),
        ],
        out_specs=out_specs,
        out_shape=out_shape,
    )


@functools.lru_cache(maxsize=None)
def _wave_call(nj):
    out_shape, out_specs = _out_specs(nj)
    full = lambda j, r: (0, 0)
    return pl.pallas_call(
        _wave_body,
        grid=(nj, NRB),
        in_specs=[
            pl.BlockSpec((1, RB, C), lambda j, r: (j, r, 0)),
            pl.BlockSpec((1, RB, C), lambda j, r: (j, r, 0)),
            pl.BlockSpec((1, N_E, C), lambda j, r: (j, 0, 0)),
            pl.BlockSpec((C, HID), full),
            pl.BlockSpec((C, HID), full),
            pl.BlockSpec((1, HID), full),
            pl.BlockSpec((1, HID), full),
            pl.BlockSpec((1, HID), full),
            pl.BlockSpec((HID, C), full),
            pl.BlockSpec((1, C), full),
        ],
        out_specs=out_specs,
        out_shape=out_shape,
    )


def kernel(z, emb, W1, b1, g_ln, b_ln, W2, b2):
    zt = jnp.transpose(z, (2, 0, 1, 3)).reshape(V, R, C)
    W1a, W1b = W1[:C], W1[C:]
    b1r = b1.reshape(1, HID)
    gr = g_ln.reshape(1, HID)
    blr = b_ln.reshape(1, HID)
    b2r = b2.reshape(1, C)

    e_all = [None] * V
    i_all = [None] * V
    loss_sum = jnp.float32(0.0)
    for w, wave in enumerate(WAVES):
        joints = jnp.array([j for j, _ in wave])
        nj = len(wave)
        zw = zt[joints]
        Ew = emb[joints]
        if w == 0:
            e, idx, lp = _root_call(nj)(zw, Ew)
        else:
            pw = jnp.stack([e_all[p] for _, p in wave])
            e, idx, lp = _wave_call(nj)(pw, zw, Ew, W1a, W1b, b1r, gr,
                                        blr, W2, b2r)
        idx = idx.reshape(nj, R)
        for k, (j, _) in enumerate(wave):
            e_all[j] = e[k]
            i_all[j] = idx[k]
        loss_sum = loss_sum + jnp.sum(lp)

    z_q = jnp.stack(e_all, axis=0).reshape(V, B, T, C).transpose(1, 2, 0, 3)
    indices = jnp.stack(i_all, axis=0).reshape(V, B, T).transpose(1, 2, 0)
    total = (1.0 + BETA) * loss_sum / (V * R * C)
    return z_q, total, indices
